# initial kernel scaffold (unmeasured)
import jax
import jax.numpy as jnp
from jax import lax
from jax.experimental import pallas as pl
from jax.experimental.pallas import tpu as pltpu

N_DEV = 32


def kernel(x, w_mat, scale_x, scale_w):
    m_per, k = x.shape
    n_tot = w_mat.shape[1]
    n_per = n_tot // N_DEV

    def body(x_ref, w_ref, sx_ref, sw_ref, out_ref, ybuf, send_sems, recv_sems):
        my = lax.axis_index("i")
        scale = sx_ref[0] * sw_ref[0]

        acc = jnp.dot(x_ref[:, :], w_ref[:, :], preferred_element_type=jnp.float32)
        y = jnp.maximum(acc * scale, 0.0)
        for j in range(N_DEV):
            ybuf[j, :, :] = y[:, j * n_per:(j + 1) * n_per]

        my_rows = pl.ds(my * m_per, m_per)
        out_ref[my_rows, :] = ybuf[my, :, :]

        rdmas = []
        for step in range(1, N_DEV):
            dst = lax.rem(my + step, N_DEV)
            rdma = pltpu.make_async_remote_copy(
                src_ref=ybuf.at[dst],
                dst_ref=out_ref.at[my_rows, :],
                send_sem=send_sems.at[step - 1],
                recv_sem=recv_sems.at[step - 1],
                device_id=(dst,),
                device_id_type=pl.DeviceIdType.MESH,
            )
            rdma.start()
            rdmas.append(rdma)
        for rdma in rdmas:
            rdma.wait()

    return pl.pallas_call(
        body,
        out_shape=jax.ShapeDtypeStruct((N_DEV * m_per, n_per), jnp.float32),
        in_specs=[
            pl.BlockSpec(memory_space=pltpu.VMEM),
            pl.BlockSpec(memory_space=pltpu.VMEM),
            pl.BlockSpec(memory_space=pltpu.SMEM),
            pl.BlockSpec(memory_space=pltpu.SMEM),
        ],
        out_specs=pl.BlockSpec(memory_space=pltpu.VMEM),
        scratch_shapes=[
            pltpu.VMEM((N_DEV, m_per, n_per), jnp.float32),
            pltpu.SemaphoreType.DMA((N_DEV - 1,)),
            pltpu.SemaphoreType.DMA((N_DEV - 1,)),
        ],
    )(x, w_mat, scale_x, scale_w)


# baseline (device time: 54204 ns/iter reference)
import jax
import jax.numpy as jnp
from jax import lax
from jax.experimental import pallas as pl
from jax.experimental.pallas import tpu as pltpu

N_DEV = 32


def kernel(x, w_mat, scale_x, scale_w):
    m_per, k = x.shape
    n_tot = w_mat.shape[1]
    n_per = n_tot // N_DEV

    def body(x_ref, w_ref, sx_ref, sw_ref, out_ref, ybuf, send_sems, recv_sems):
        my = lax.axis_index("i")
        scale = sx_ref[0] * sw_ref[0]

        x8 = x_ref[:, :].astype(jnp.float8_e4m3fn)
        w8 = w_ref[:, :].astype(jnp.float8_e4m3fn)
        acc = jnp.dot(x8, w8, preferred_element_type=jnp.float32)
        y = jnp.maximum(acc * scale, 0.0)
        for j in range(N_DEV):
            ybuf[j, :, :] = y[:, j * n_per:(j + 1) * n_per]

        my_rows = pl.ds(my * m_per, m_per)
        out_ref[my_rows, :] = ybuf[my, :, :]

        rdmas = []
        for step in range(1, N_DEV):
            dst = lax.rem(my + step, N_DEV)
            rdma = pltpu.make_async_remote_copy(
                src_ref=ybuf.at[dst],
                dst_ref=out_ref.at[my_rows, :],
                send_sem=send_sems.at[step - 1],
                recv_sem=recv_sems.at[step - 1],
                device_id=(dst,),
                device_id_type=pl.DeviceIdType.MESH,
            )
            rdma.start()
            rdmas.append(rdma)
        for rdma in rdmas:
            rdma.wait()

    return pl.pallas_call(
        body,
        out_shape=jax.ShapeDtypeStruct((N_DEV * m_per, n_per), jnp.float32),
        in_specs=[
            pl.BlockSpec(memory_space=pltpu.VMEM),
            pl.BlockSpec(memory_space=pltpu.VMEM),
            pl.BlockSpec(memory_space=pltpu.SMEM),
            pl.BlockSpec(memory_space=pltpu.SMEM),
        ],
        out_specs=pl.BlockSpec(memory_space=pltpu.VMEM),
        scratch_shapes=[
            pltpu.VMEM((N_DEV, m_per, n_per), jnp.float32),
            pltpu.SemaphoreType.DMA((N_DEV - 1,)),
            pltpu.SemaphoreType.DMA((N_DEV - 1,)),
        ],
        compiler_params=pltpu.CompilerParams(
            vmem_limit_bytes=100 * 1024 * 1024,
        ),
    )(x, w_mat, scale_x, scale_w)


# device time: 27994 ns/iter; 1.9363x vs baseline; 1.9363x over previous
import jax
import jax.numpy as jnp
from jax import lax
from jax.experimental import pallas as pl
from jax.experimental.pallas import tpu as pltpu

N_DEV = 32
NCHUNK = 4
BLOCKS_PER_CHUNK = N_DEV // NCHUNK


def kernel(x, w_mat, scale_x, scale_w):
    m_per, k = x.shape
    n_tot = w_mat.shape[1]
    n_per = n_tot // N_DEV
    n_chunk = n_tot // NCHUNK

    def body(x_ref, w_ref, sx_ref, sw_ref, out_ref, wv, ybuf, rbuf,
             dma_sems, send_sems, recv_sems):
        my = lax.axis_index("i")

        bsem = pltpu.get_barrier_semaphore()
        for d in range(N_DEV):
            pl.semaphore_signal(
                bsem, inc=1, device_id=(d,),
                device_id_type=pl.DeviceIdType.MESH,
            )

        wdmas = []
        for c in range(NCHUNK):
            cc = lax.rem(c + (my // BLOCKS_PER_CHUNK) + 1, NCHUNK)
            dma = pltpu.make_async_copy(
                w_ref.at[:, pl.ds(cc * n_chunk, n_chunk)],
                wv.at[c],
                dma_sems.at[c],
            )
            dma.start()
            wdmas.append((dma, cc))

        scale = sx_ref[0] * sw_ref[0]
        x8 = x_ref[:, :].astype(jnp.float8_e4m3fn)

        barrier_done = False
        rdmas = [None] * N_DEV
        for c in range(NCHUNK):
            dma, cc = wdmas[c]
            dma.wait()
            w8 = wv[c, :, :].astype(jnp.float8_e4m3fn)
            acc = jnp.dot(x8, w8, preferred_element_type=jnp.float32)
            y = jnp.maximum(acc * scale, 0.0).astype(jnp.bfloat16)
            for jj in range(BLOCKS_PER_CHUNK):
                ybuf[cc * BLOCKS_PER_CHUNK + jj, :, :] = (
                    y[:, jj * n_per:(jj + 1) * n_per]
                )
            if not barrier_done:
                pl.semaphore_wait(bsem, N_DEV)
                barrier_done = True
            for jj in range(BLOCKS_PER_CHUNK):
                d = cc * BLOCKS_PER_CHUNK + lax.rem(
                    my + 1 + jj, BLOCKS_PER_CHUNK
                )
                step = lax.rem(d - my + N_DEV, N_DEV)

                @pl.when(step != 0)
                def _():
                    rdma = pltpu.make_async_remote_copy(
                        src_ref=ybuf.at[d],
                        dst_ref=rbuf.at[my],
                        send_sem=send_sems.at[step - 1],
                        recv_sem=recv_sems.at[step - 1],
                        device_id=(d,),
                        device_id_type=pl.DeviceIdType.MESH,
                    )
                    rdma.start()

                @pl.when(step == 0)
                def _():
                    out_ref[pl.ds(my * m_per, m_per), :] = (
                        ybuf[d, :, :].astype(jnp.float32)
                    )

        for s in range(N_DEV - 1):
            src = lax.rem(my - s - 1 + N_DEV, N_DEV)
            recv = pltpu.make_async_remote_copy(
                src_ref=ybuf.at[0],
                dst_ref=rbuf.at[src],
                send_sem=send_sems.at[s],
                recv_sem=recv_sems.at[s],
                device_id=(my,),
                device_id_type=pl.DeviceIdType.MESH,
            )
            recv.wait_recv()
            out_ref[pl.ds(src * m_per, m_per), :] = (
                rbuf[src, :, :].astype(jnp.float32)
            )
        for s in range(N_DEV - 1):
            snd = pltpu.make_async_remote_copy(
                src_ref=ybuf.at[0],
                dst_ref=rbuf.at[0],
                send_sem=send_sems.at[s],
                recv_sem=recv_sems.at[s],
                device_id=(my,),
                device_id_type=pl.DeviceIdType.MESH,
            )
            snd.wait_send()

    return pl.pallas_call(
        body,
        out_shape=jax.ShapeDtypeStruct((N_DEV * m_per, n_per), jnp.float32),
        in_specs=[
            pl.BlockSpec(memory_space=pltpu.VMEM),
            pl.BlockSpec(memory_space=pltpu.MemorySpace.HBM),
            pl.BlockSpec(memory_space=pltpu.SMEM),
            pl.BlockSpec(memory_space=pltpu.SMEM),
        ],
        out_specs=pl.BlockSpec(memory_space=pltpu.VMEM),
        scratch_shapes=[
            pltpu.VMEM((NCHUNK, k, n_tot // NCHUNK), jnp.float32),
            pltpu.VMEM((N_DEV, m_per, n_per), jnp.bfloat16),
            pltpu.VMEM((N_DEV, m_per, n_per), jnp.bfloat16),
            pltpu.SemaphoreType.DMA((NCHUNK,)),
            pltpu.SemaphoreType.DMA((N_DEV - 1,)),
            pltpu.SemaphoreType.DMA((N_DEV - 1,)),
        ],
        compiler_params=pltpu.CompilerParams(
            vmem_limit_bytes=100 * 1024 * 1024,
            collective_id=0,
        ),
    )(x, w_mat, scale_x, scale_w)
